# Initial kernel scaffold; baseline (speedup 1.0000x reference)
import jax
import jax.numpy as jnp
from jax.experimental import pallas as pl


def kernel(x, edge_index, edge_attr, batch, W_src1, W_dst1, W_edge1, Wm1_1, gamma1, beta1, Wm2_1, W_src2, W_dst2, W_edge2, Wm1_2, gamma2, beta2, Wm2_2, W_src3, W_dst3, W_edge3, Wm1_3, gamma3, beta3, Wm2_3):
    raise NotImplementedError("write your pallas kernel here")


# speedup vs baseline: 2.8937x; 2.8937x over previous
import functools

import jax
import jax.numpy as jnp
from jax import lax
from jax.experimental import pallas as pl
from jax.experimental.pallas import tpu as pltpu
from jax.experimental.pallas import tpu_sc as plsc

EPS = 1e-7
N_NODES = 10000
N_EDGES = 160000
N_GRAPHS = 64
D_FEAT = 256

NC, NS = 2, 16
NW = NC * NS
CB = 64
EC = 128
E_PAD = 163840
EPT = E_PAD // NS
NCHUNK = EPT // EC
NPAD = 10240
RPT = NPAD // NS
NB = 400
EB = 2048


def _dot(a, b):
    return lax.dot_general(a, b, (((1,), (0,)), ((), ())),
                           precision=lax.Precision.HIGHEST,
                           preferred_element_type=jnp.float32)


def _edge_proj_body(ea_ref, w1_ref, w2_ref, w3_ref, e1_ref, e2_ref, e3_ref):
    ea = ea_ref[...]
    for w_ref, e_ref in ((w1_ref, e1_ref), (w2_ref, e2_ref), (w3_ref, e3_ref)):
        w = w_ref[...]
        for q in range(e_ref.shape[0]):
            e_ref[q] = _dot(ea, w[:, q * CB:(q + 1) * CB])


def _proj_body(x_ref, ws_ref, wd_ref, hs_ref, hd_ref):
    xb = x_ref[...]
    ws = ws_ref[...]
    for q in range(hs_ref.shape[0]):
        hs_ref[q] = _dot(xb, ws[:, q * CB:(q + 1) * CB])
    hd_ref[...] = _dot(xb, wd_ref[...])


def _mlp1_body(agg_ref, hd_ref, wm1_ref, h1_ref, st_ref):
    nc = agg_ref.shape[0]
    out = jnp.concatenate([agg_ref[q] for q in range(nc)], axis=1) + hd_ref[...]
    h1 = _dot(out, wm1_ref[...])
    h1_ref[...] = h1

    @pl.when(pl.program_id(0) == 0)
    def _():
        st_ref[...] = jnp.zeros_like(st_ref)

    st_ref[...] += jnp.stack([jnp.sum(h1, axis=0), jnp.sum(h1 * h1, axis=0)])


def _bn_relu_mlp2(h1_ref, st_ref, gamma_ref, beta_ref, wm2_ref):
    st = st_ref[...]
    mu = st[0] / N_NODES
    var = st[1] / N_NODES - mu * mu
    rstd = lax.rsqrt(var + 1e-5)
    hn = jnp.maximum((h1_ref[...] - mu) * (rstd * gamma_ref[...]) + beta_ref[...], 0.0)
    t = _dot(hn, wm2_ref[...])
    return jnp.where(t > 0.0, t, jnp.exp(jnp.minimum(t, 0.0)) - 1.0)


def _mlp2_proj_body(h1_ref, st_ref, gamma_ref, beta_ref, wm2_ref,
                    ws_ref, wd_ref, hs_ref, hd_ref):
    h = _bn_relu_mlp2(h1_ref, st_ref, gamma_ref, beta_ref, wm2_ref)
    ws = ws_ref[...]
    for q in range(hs_ref.shape[0]):
        hs_ref[q] = _dot(h, ws[:, q * CB:(q + 1) * CB])
    hd_ref[...] = _dot(h, wd_ref[...])


def _mlp2_last_body(h1_ref, st_ref, gamma_ref, beta_ref, wm2_ref, h_ref):
    h_ref[...] = _bn_relu_mlp2(h1_ref, st_ref, gamma_ref, beta_ref, wm2_ref)


def _pool_finish_body(p_ref, out_ref):
    m = jnp.max(p_ref[...][:, :N_GRAPHS, :], axis=0)
    out_ref[...] = jnp.where(jnp.isfinite(m), m, 0.0)


def _fill(ref, rows, width, value):
    def body(r, carry):
        for k in range(width // 16):
            ref[r, pl.ds(k * 16, 16)] = jnp.full((16,), value, jnp.float32)
        return carry
    lax.fori_loop(0, rows, body, 0)


def _make_edge_sc(nc, interpret=False):
    qpc = nc // NC

    @functools.partial(
        pl.kernel,
        out_type=jax.ShapeDtypeStruct((nc, NPAD, CB), jnp.float32),
        mesh=plsc.VectorSubcoreMesh(core_axis_name="c", subcore_axis_name="s",
                                    num_cores=NC, num_subcores=NS),
        scratch_types=[
            pltpu.VMEM((1, EC), jnp.int32),
            pltpu.VMEM((1, EC), jnp.int32),
            pltpu.VMEM((EC, CB), jnp.float32),
            pltpu.VMEM((EC, CB), jnp.float32),
            pltpu.VMEM((EC, CB), jnp.float32),
            pltpu.VMEM((EC, CB), jnp.float32),
            pltpu.MemorySpace.VMEM_SHARED((NPAD, CB), jnp.float32),
            pltpu.MemorySpace.VMEM_SHARED((NPAD, CB), jnp.float32),
            pltpu.SemaphoreType.DMA,
        ],
        compiler_params=pltpu.CompilerParams(use_tc_tiling_on_sc=False),
        interpret=interpret,
    )
    def edge_kernel(hsrc, e, srcr, dstr, out,
                    idxs, idxd, g_b, e_b, wm_b, w_b, num_sh, den_sh, sem):
        c = lax.axis_index("c")
        t = lax.axis_index("s")
        for qq in range(qpc):
            q = c * qpc + qq
            _fill(wm_b, EC, CB, 0.0)
            for z in range(RPT // EC):
                pltpu.sync_copy(wm_b, num_sh.at[pl.ds(t * RPT + z * EC, EC)])
                pltpu.sync_copy(wm_b, den_sh.at[pl.ds(t * RPT + z * EC, EC)])
            plsc.subcore_barrier()

            def chunk_body(j, carry):
                base = t * EPT + j * EC
                pltpu.sync_copy(srcr.at[pl.ds(t * NCHUNK + j, 1)], idxs)
                pltpu.sync_copy(dstr.at[pl.ds(t * NCHUNK + j, 1)], idxd)
                pltpu.async_copy(hsrc.at[q].at[idxs.at[0]], g_b, sem).wait()
                pltpu.sync_copy(e.at[q].at[pl.ds(base, EC)], e_b)

                def row_body(r, carry2):
                    for k in range(CB // 16):
                        s = pl.ds(k * 16, 16)
                        msg = jnp.maximum(g_b[r, s] + e_b[r, s], 0.0) + EPS
                        w = jnp.exp(msg)
                        wm_b[r, s] = msg * w
                        w_b[r, s] = w
                    return carry2

                lax.fori_loop(0, EC, row_body, 0)
                pltpu.sync_copy(wm_b, num_sh.at[idxd.at[0]], add=True)
                pltpu.sync_copy(w_b, den_sh.at[idxd.at[0]], add=True)
                return carry

            lax.fori_loop(0, NCHUNK, chunk_body, 0)
            plsc.subcore_barrier()

            for z in range(RPT // EC):
                rbase = t * RPT + z * EC
                pltpu.sync_copy(num_sh.at[pl.ds(rbase, EC)], g_b)
                pltpu.sync_copy(den_sh.at[pl.ds(rbase, EC)], e_b)

                def fin_body(r, carry2):
                    for k in range(CB // 16):
                        s = pl.ds(k * 16, 16)
                        wm_b[r, s] = g_b[r, s] / (e_b[r, s] + 1e-16)
                    return carry2

                lax.fori_loop(0, EC, fin_body, 0)
                pltpu.sync_copy(wm_b, out.at[q].at[pl.ds(rbase, EC)])

    return edge_kernel


def _make_pool_sc(interpret=False):
    npt = NPAD // NW

    @functools.partial(
        pl.kernel,
        out_type=jax.ShapeDtypeStruct((NW, N_GRAPHS + 1, 128), jnp.float32),
        mesh=plsc.VectorSubcoreMesh(core_axis_name="c", subcore_axis_name="s",
                                    num_cores=NC, num_subcores=NS),
        scratch_types=[
            pltpu.VMEM((npt,), jnp.int32),
            pltpu.VMEM((npt, 128), jnp.float32),
            pltpu.VMEM((N_GRAPHS + 1, 128), jnp.float32),
        ],
        compiler_params=pltpu.CompilerParams(use_tc_tiling_on_sc=False),
        interpret=interpret,
    )
    def pool_kernel(h, batchr, out, b_v, h_v, acc):
        c = lax.axis_index("c")
        t = lax.axis_index("s")
        w = t * NC + c
        base = w * npt
        pltpu.sync_copy(batchr.at[pl.ds(base, npt)], b_v)
        pltpu.sync_copy(h.at[pl.ds(base, npt)], h_v)
        _fill(acc, N_GRAPHS + 1, 128, float("-inf"))

        def body(gi, carry):
            bvec = b_v[pl.ds(gi * 16, 16)]
            for j in range(16):
                b = bvec[j]
                i = gi * 16 + j
                for k in range(8):
                    s = pl.ds(k * 16, 16)
                    acc[b, s] = jnp.maximum(acc[b, s], h_v[i, s])
            return carry

        lax.fori_loop(0, npt // 16, body, 0)
        pltpu.sync_copy(acc, out.at[w])

    return pool_kernel


def _build(interpret=False):
    k = {}

    def tc(body, grid, in_specs, out_shape, out_specs):
        return pl.pallas_call(body, grid=grid, in_specs=in_specs,
                              out_shape=out_shape, out_specs=out_specs,
                              interpret=interpret)

    full = lambda shape: pl.BlockSpec(shape, lambda i: (0,) * len(shape))

    k["edge_proj"] = tc(
        _edge_proj_body, (E_PAD // EB,),
        [pl.BlockSpec((EB, 16), lambda i: (i, 0)),
         full((16, 128)), full((16, 256)), full((16, 128))],
        (jax.ShapeDtypeStruct((2, E_PAD, CB), jnp.float32),
         jax.ShapeDtypeStruct((4, E_PAD, CB), jnp.float32),
         jax.ShapeDtypeStruct((2, E_PAD, CB), jnp.float32)),
        (pl.BlockSpec((2, EB, CB), lambda i: (0, i, 0)),
         pl.BlockSpec((4, EB, CB), lambda i: (0, i, 0)),
         pl.BlockSpec((2, EB, CB), lambda i: (0, i, 0))),
    )

    def proj(cin, cout):
        nc = cout // CB
        return tc(
            _proj_body, (N_NODES // NB,),
            [pl.BlockSpec((NB, cin), lambda i: (i, 0)),
             full((cin, cout)), full((cin, cout))],
            (jax.ShapeDtypeStruct((nc, N_NODES, CB), jnp.float32),
             jax.ShapeDtypeStruct((N_NODES, cout), jnp.float32)),
            (pl.BlockSpec((nc, NB, CB), lambda i: (0, i, 0)),
             pl.BlockSpec((NB, cout), lambda i: (i, 0))),
        )

    k["proj1"] = proj(D_FEAT, 128)

    def mlp1(cout):
        nc = cout // CB
        return tc(
            _mlp1_body, (N_NODES // NB,),
            [pl.BlockSpec((nc, NB, CB), lambda i: (0, i, 0)),
             pl.BlockSpec((NB, cout), lambda i: (i, 0)),
             full((cout, 2 * cout))],
            (jax.ShapeDtypeStruct((N_NODES, 2 * cout), jnp.float32),
             jax.ShapeDtypeStruct((2, 2 * cout), jnp.float32)),
            (pl.BlockSpec((NB, 2 * cout), lambda i: (i, 0)),
             pl.BlockSpec((2, 2 * cout), lambda i: (0, 0))),
        )

    k["mlp1_128"] = mlp1(128)
    k["mlp1_256"] = mlp1(256)

    def mlp2_proj(cout, cout2):
        nc2 = cout2 // CB
        return tc(
            _mlp2_proj_body, (N_NODES // NB,),
            [pl.BlockSpec((NB, 2 * cout), lambda i: (i, 0)),
             full((2, 2 * cout)), full((2 * cout,)), full((2 * cout,)),
             full((2 * cout, cout)), full((cout, cout2)), full((cout, cout2))],
            (jax.ShapeDtypeStruct((nc2, N_NODES, CB), jnp.float32),
             jax.ShapeDtypeStruct((N_NODES, cout2), jnp.float32)),
            (pl.BlockSpec((nc2, NB, CB), lambda i: (0, i, 0)),
             pl.BlockSpec((NB, cout2), lambda i: (i, 0))),
        )

    k["mlp2_proj_1"] = mlp2_proj(128, 256)
    k["mlp2_proj_2"] = mlp2_proj(256, 128)

    k["mlp2_last"] = tc(
        _mlp2_last_body, (N_NODES // NB,),
        [pl.BlockSpec((NB, 256), lambda i: (i, 0)),
         full((2, 256)), full((256,)), full((256,)), full((256, 128))],
        jax.ShapeDtypeStruct((NPAD, 128), jnp.float32),
        pl.BlockSpec((NB, 128), lambda i: (i, 0)),
    )

    k["pool_finish"] = tc(
        _pool_finish_body, (1,),
        [full((NW, N_GRAPHS + 1, 128))],
        jax.ShapeDtypeStruct((N_GRAPHS, 128), jnp.float32),
        full((N_GRAPHS, 128)),
    )

    k["edge_sc2"] = _make_edge_sc(2, interpret)
    k["edge_sc4"] = _make_edge_sc(4, interpret)
    k["pool_sc"] = _make_pool_sc(interpret)
    return k


@functools.cache
def _kernels():
    return _build()


def kernel(x, edge_index, edge_attr, batch,
           W_src1, W_dst1, W_edge1, Wm1_1, gamma1, beta1, Wm2_1,
           W_src2, W_dst2, W_edge2, Wm1_2, gamma2, beta2, Wm2_2,
           W_src3, W_dst3, W_edge3, Wm1_3, gamma3, beta3, Wm2_3):
    src = edge_index[0].astype(jnp.int32)
    dst = edge_index[1].astype(jnp.int32)
    pad = E_PAD - N_EDGES
    src_p = jnp.concatenate([src, jnp.zeros((pad,), jnp.int32)]
                            ).reshape(E_PAD // EC, EC)
    dst_p = jnp.concatenate([dst, jnp.full((pad,), N_NODES, jnp.int32)]
                            ).reshape(E_PAD // EC, EC)
    ea_p = jnp.concatenate([edge_attr, jnp.zeros((pad, 16), jnp.float32)])
    batch_p = jnp.concatenate([batch.astype(jnp.int32),
                               jnp.full((NPAD - N_NODES,), N_GRAPHS, jnp.int32)])

    _K = _kernels()
    e1, e2, e3 = _K["edge_proj"](ea_p, W_edge1, W_edge2, W_edge3)

    hs, hd = _K["proj1"](x, W_src1, W_dst1)
    agg = _K["edge_sc2"](hs, e1, src_p, dst_p)
    h1, st = _K["mlp1_128"](agg, hd, Wm1_1)
    hs, hd = _K["mlp2_proj_1"](h1, st, gamma1, beta1, Wm2_1, W_src2, W_dst2)

    agg = _K["edge_sc4"](hs, e2, src_p, dst_p)
    h1, st = _K["mlp1_256"](agg, hd, Wm1_2)
    hs, hd = _K["mlp2_proj_2"](h1, st, gamma2, beta2, Wm2_2, W_src3, W_dst3)

    agg = _K["edge_sc2"](hs, e3, src_p, dst_p)
    h1, st = _K["mlp1_128"](agg, hd, Wm1_3)
    h3 = _K["mlp2_last"](h1, st, gamma3, beta3, Wm2_3)

    part = _K["pool_sc"](h3, batch_p)
    return _K["pool_finish"](part)
